# Initial kernel scaffold; baseline (speedup 1.0000x reference)
import jax
import jax.numpy as jnp
from jax.experimental import pallas as pl


def kernel(node_feats, edge_feats, edge_index, W1, b1, W2, b2, We1, be1, We2, be2, conv_b, Wih, bih, Whh, bhh):
    raise NotImplementedError("write your pallas kernel here")


# speedup vs baseline: 1.0761x; 1.0761x over previous
import functools

import jax
import jax.numpy as jnp
from jax import lax
from jax.experimental import pallas as pl
from jax.experimental.pallas import tpu as pltpu
from jax.experimental.pallas import tpu_sc as plsc

_NC = 2
_NS = 16
_NW = _NC * _NS

_F32 = jnp.float32


_BF16 = jnp.bfloat16


def _proj_body(nf_ref, w1_ref, b1_ref, w2_ref, b2_ref, o_ref):
    x = jnp.dot(nf_ref[...].astype(_BF16), w1_ref[...],
                preferred_element_type=_F32)
    x = jnp.maximum(x + b1_ref[...], 0.0)
    o_ref[...] = jnp.dot(x.astype(_BF16), w2_ref[...],
                         preferred_element_type=_F32) + b2_ref[...]


def _edge1_body(ef_ref, we1_ref, be1_ref, o_ref):
    x = jnp.dot(ef_ref[...].astype(_BF16), we1_ref[...],
                preferred_element_type=_F32)
    o_ref[...] = jnp.maximum(x + be1_ref[...], 0.0)


def _msg_body(h_dim, a_ref, hs_ref, we2_ref, be2_ref, o_ref):
    ew = jnp.dot(a_ref[...].astype(_BF16), we2_ref[...],
                 preferred_element_type=_F32) + be2_ref[...]
    ew = ew.astype(_BF16).astype(_F32)
    hs = hs_ref[...].astype(_BF16).astype(_F32)
    acc = hs[:, 0:1] * ew[:, 0:h_dim]
    for i in range(1, h_dim):
        acc = acc + hs[:, i : i + 1] * ew[:, i * h_dim : (i + 1) * h_dim]
    o_ref[...] = acc


def _gru_body(agg2_ref, cb_ref, hid_ref, wi_ref, bi_ref, wh_ref, bh_ref, o_ref):
    node = jnp.maximum(agg2_ref[0] + agg2_ref[1] + cb_ref[...], 0.0)
    hid = hid_ref[...]
    nb = node.astype(_BF16)
    hb = hid.astype(_BF16)
    gi_r = jnp.dot(nb, wi_ref[0], preferred_element_type=_F32) + bi_ref[0]
    gi_z = jnp.dot(nb, wi_ref[1], preferred_element_type=_F32) + bi_ref[1]
    gi_n = jnp.dot(nb, wi_ref[2], preferred_element_type=_F32) + bi_ref[2]
    gh_r = jnp.dot(hb, wh_ref[0], preferred_element_type=_F32) + bh_ref[0]
    gh_z = jnp.dot(hb, wh_ref[1], preferred_element_type=_F32) + bh_ref[1]
    gh_n = jnp.dot(hb, wh_ref[2], preferred_element_type=_F32) + bh_ref[2]
    r = jax.nn.sigmoid(gi_r + gh_r)
    z = jax.nn.sigmoid(gi_z + gh_z)
    n = jnp.tanh(gi_n + r * gh_n)
    o_ref[...] = (1.0 - z) * n + z * hid


def _gather_body(chunk, nchunk, h_hbm, src_hbm, out_hbm, idx_v, rows_v, sem):
    c = lax.axis_index("c")
    s = lax.axis_index("s")
    wid = s * _NC + c
    base = wid * (chunk * nchunk)
    for j in range(nchunk):
        off = base + j * chunk
        pltpu.sync_copy(src_hbm.at[pl.ds(off, chunk)], idx_v)
        pltpu.async_copy(h_hbm.at[idx_v], rows_v, sem).wait()
        pltpu.sync_copy(rows_v, out_hbm.at[pl.ds(off, chunk)])


def _scatter_body(chunk, nchunk, msg_hbm, dst_hbm, zero_hbm, out_hbm,
                  idx_v, rows_v, acc_sh):
    c = lax.axis_index("c")
    s = lax.axis_index("s")
    base = (c * _NS + s) * (chunk * nchunk)

    @pl.when(s == 0)
    def _zero():
        pltpu.sync_copy(zero_hbm, acc_sh)

    plsc.subcore_barrier()
    for j in range(nchunk):
        off = base + j * chunk
        pltpu.sync_copy(dst_hbm.at[pl.ds(off, chunk)], idx_v)
        pltpu.sync_copy(msg_hbm.at[pl.ds(off, chunk)], rows_v)
        pltpu.sync_copy(rows_v, acc_sh.at[idx_v], add=True)
    plsc.subcore_barrier()

    @pl.when(s == 0)
    def _flush():
        pltpu.sync_copy(acc_sh, out_hbm.at[c])


def kernel(node_feats, edge_feats, edge_index, W1, b1, W2, b2,
           We1, be1, We2, be2, conv_b, Wih, bih, Whh, bhh):
    n_nodes, node_in = node_feats.shape
    n_edges, edge_in = edge_feats.shape
    eh = We1.shape[1]
    h_dim = W1.shape[1]
    steps = 3

    src = edge_index[0]
    dst = edge_index[1]

    W1b = W1.astype(_BF16)
    W2b = W2.astype(_BF16)
    We1b = We1.astype(_BF16)
    We2b = We2.astype(_BF16)
    be2r = be2.reshape(1, h_dim * h_dim)
    b1r = b1.reshape(1, h_dim)
    b2r = b2.reshape(1, h_dim)
    be1r = be1.reshape(1, eh)
    cbr = conv_b.reshape(1, h_dim)
    Wi3 = Wih.reshape(h_dim, 3, h_dim).transpose(1, 0, 2).astype(_BF16)
    Wh3 = Whh.reshape(h_dim, 3, h_dim).transpose(1, 0, 2).astype(_BF16)
    bi3 = bih.reshape(3, 1, h_dim)
    bh3 = bhh.reshape(3, 1, h_dim)
    zeros_nh = jnp.zeros((n_nodes, h_dim), _F32)

    proj = pl.pallas_call(
        _proj_body,
        out_shape=jax.ShapeDtypeStruct((n_nodes, h_dim), _F32),
    )

    ba = 8000
    edge1 = pl.pallas_call(
        _edge1_body,
        grid=(n_edges // ba,),
        in_specs=[
            pl.BlockSpec((ba, edge_in), lambda i: (i, 0)),
            pl.BlockSpec((edge_in, eh), lambda i: (0, 0)),
            pl.BlockSpec((1, eh), lambda i: (0, 0)),
        ],
        out_specs=pl.BlockSpec((ba, eh), lambda i: (i, 0)),
        out_shape=jax.ShapeDtypeStruct((n_edges, eh), _F32),
    )

    bm = 1000
    msgk = pl.pallas_call(
        functools.partial(_msg_body, h_dim),
        grid=(n_edges // bm,),
        in_specs=[
            pl.BlockSpec((bm, eh), lambda i: (i, 0)),
            pl.BlockSpec((bm, h_dim), lambda i: (i, 0)),
            pl.BlockSpec((eh, h_dim * h_dim), lambda i: (0, 0)),
            pl.BlockSpec((1, h_dim * h_dim), lambda i: (0, 0)),
        ],
        out_specs=pl.BlockSpec((bm, h_dim), lambda i: (i, 0)),
        out_shape=jax.ShapeDtypeStruct((n_edges, h_dim), _F32),
    )

    gru = pl.pallas_call(
        _gru_body,
        out_shape=jax.ShapeDtypeStruct((n_nodes, h_dim), _F32),
    )

    chunk = 1000
    nchunk = n_edges // (_NW * chunk)
    mesh = plsc.VectorSubcoreMesh(core_axis_name="c", subcore_axis_name="s")
    sc_params = pltpu.CompilerParams(use_tc_tiling_on_sc=False)
    gather = pl.kernel(
        functools.partial(_gather_body, chunk, nchunk),
        out_type=jax.ShapeDtypeStruct((n_edges, h_dim), _F32),
        mesh=mesh,
        compiler_params=sc_params,
        scratch_types=[
            pltpu.VMEM((chunk,), jnp.int32),
            pltpu.VMEM((chunk, h_dim), _F32),
            pltpu.SemaphoreType.DMA,
        ],
    )

    scatter = pl.kernel(
        functools.partial(_scatter_body, chunk, nchunk),
        out_type=jax.ShapeDtypeStruct((_NC, n_nodes, h_dim), _F32),
        mesh=mesh,
        compiler_params=sc_params,
        scratch_types=[
            pltpu.VMEM((chunk,), jnp.int32),
            pltpu.VMEM((chunk, h_dim), _F32),
            pltpu.VMEM_SHARED((n_nodes, h_dim), _F32),
        ],
    )

    h = proj(node_feats, W1b, b1r, W2b, b2r)
    a = edge1(edge_feats, We1b, be1r)
    hidden = h
    for _ in range(steps):
        hs = gather(h, src)
        msg = msgk(a, hs, We2b, be2r)
        agg2 = scatter(msg, dst, zeros_nh)
        hidden = gru(agg2, cbr, hidden, Wi3, bi3, Wh3, bh3)
        h = hidden
    return h
